# Initial kernel scaffold; baseline (speedup 1.0000x reference)
import jax
import jax.numpy as jnp
from jax.experimental import pallas as pl


def kernel(x, te):
    raise NotImplementedError("write your pallas kernel here")


# speedup vs baseline: 4.0932x; 4.0932x over previous
import functools
import jax
import jax.numpy as jnp
from jax import lax
from jax.experimental import pallas as pl
from jax.experimental.pallas import tpu as pltpu
from jax.experimental.pallas import tpu_sc as plsc

D_MODEL = 64
NUM_CORES = 2
NUM_SUBCORES = 16
NUM_WORKERS = NUM_CORES * NUM_SUBCORES
CHUNK = 800


def _gather_body(n_per_w, x_hbm, te_hbm, out_hbm, idx_v, rows_v, gsem):
    wid = lax.axis_index("s") * NUM_CORES + lax.axis_index("c")
    nchunks = n_per_w // CHUNK

    def body(g, carry):
        base = wid * n_per_w + g * CHUNK
        pltpu.sync_copy(x_hbm.at[pl.ds(base, CHUNK)], idx_v)
        pltpu.async_copy(te_hbm.at[idx_v], rows_v, gsem).wait()
        pltpu.sync_copy(rows_v, out_hbm.at[pl.ds(base, CHUNK)])
        return carry

    lax.fori_loop(0, nchunks, body, 0, unroll=False)


def kernel(x, te):
    batch, seq = x.shape
    n_total = batch * seq
    n_per_w = n_total // NUM_WORKERS
    assert n_per_w % CHUNK == 0

    x_flat = x.reshape(n_total).astype(jnp.int32)

    mesh = plsc.VectorSubcoreMesh(core_axis_name="c", subcore_axis_name="s")
    run = pl.kernel(
        functools.partial(_gather_body, n_per_w),
        out_type=jax.ShapeDtypeStruct((n_total, D_MODEL), jnp.float32),
        mesh=mesh,
        scratch_types=[
            pltpu.VMEM((CHUNK,), jnp.int32),
            pltpu.VMEM((CHUNK, D_MODEL), jnp.float32),
            pltpu.SemaphoreType.DMA,
        ],
        compiler_params=pltpu.CompilerParams(use_tc_tiling_on_sc=False),
    )
    out = run(x_flat, te)
    return out.reshape(batch, seq, D_MODEL)


# speedup vs baseline: 4.2556x; 1.0397x over previous
import functools
import jax
import jax.numpy as jnp
from jax import lax
from jax.experimental import pallas as pl
from jax.experimental.pallas import tpu as pltpu
from jax.experimental.pallas import tpu_sc as plsc

D_MODEL = 64
NUM_CORES = 2
NUM_SUBCORES = 16
NUM_WORKERS = NUM_CORES * NUM_SUBCORES
CHUNK = 400
NBUF = 4


def _gather_body(n_per_w, x_hbm, te_hbm, out_hbm, idx_v, rows_v, gsem, wsem):
    nchunks = n_per_w // CHUNK
    wid = lax.axis_index("s") * NUM_CORES + lax.axis_index("c")
    base = wid * n_per_w
    pltpu.sync_copy(x_hbm.at[pl.ds(base, n_per_w)], idx_v)

    def idx_slice(g):
        return idx_v.at[pl.ds(g * CHUNK, CHUNK)]

    def out_slice(g):
        return out_hbm.at[pl.ds(base + g * CHUNK, CHUNK)]

    def start_gather(g, b):
        pltpu.async_copy(te_hbm.at[idx_slice(g)], rows_v.at[b], gsem.at[b])

    def wait_gather(g, b):
        pltpu.make_async_copy(te_hbm.at[idx_slice(g)], rows_v.at[b],
                              gsem.at[b]).wait()

    def start_wo(g, b):
        pltpu.async_copy(rows_v.at[b], out_slice(g), wsem.at[b])

    def wait_wo(g, b):
        pltpu.make_async_copy(rows_v.at[b], out_slice(g), wsem.at[b]).wait()

    for b in range(NBUF):
        start_gather(b, b)
    for g in (0, 1):
        wait_gather(g, g)
        start_wo(g, g)
    for g in (2, 3):
        wait_gather(g, g)
        start_wo(g, g)
        wait_wo(g - 2, g - 2)
        start_gather(g + 2, g - 2)

    def outer(go, carry):
        for b in range(NBUF):
            g = go * NBUF + b
            wait_gather(g, b)
            start_wo(g, b)
            b2 = (b + 2) % NBUF
            wait_wo(g - 2, b2)
            start_gather(g + 2, b2)
        return carry

    lax.fori_loop(1, nchunks // NBUF - 1, outer, 0, unroll=False)

    for b in range(NBUF):
        g = nchunks - NBUF + b
        wait_gather(g, b)
        start_wo(g, b)
        if b < 2:
            b2 = (b + 2) % NBUF
            wait_wo(g - 2, b2)
            start_gather(g + 2, b2)
    for b in range(NBUF):
        wait_wo(nchunks - NBUF + b, b)


def kernel(x, te):
    batch, seq = x.shape
    n_total = batch * seq
    n_per_w = n_total // NUM_WORKERS
    assert n_per_w % (NBUF * CHUNK) == 0

    x_flat = x.reshape(n_total).astype(jnp.int32)

    mesh = plsc.VectorSubcoreMesh(core_axis_name="c", subcore_axis_name="s")
    run = pl.kernel(
        functools.partial(_gather_body, n_per_w),
        out_type=jax.ShapeDtypeStruct((n_total, D_MODEL), jnp.float32),
        mesh=mesh,
        scratch_types=[
            pltpu.VMEM((n_per_w,), jnp.int32),
            pltpu.VMEM((NBUF, CHUNK, D_MODEL), jnp.float32),
            pltpu.SemaphoreType.DMA((NBUF,)),
            pltpu.SemaphoreType.DMA((NBUF,)),
        ],
        compiler_params=pltpu.CompilerParams(use_tc_tiling_on_sc=False),
    )
    out = run(x_flat, te)
    return out.reshape(batch, seq, D_MODEL)


# speedup vs baseline: 4.2619x; 1.0015x over previous
import functools
import jax
import jax.numpy as jnp
from jax import lax
from jax.experimental import pallas as pl
from jax.experimental.pallas import tpu as pltpu
from jax.experimental.pallas import tpu_sc as plsc

D_MODEL = 64
NUM_CORES = 2
NUM_SUBCORES = 16
NUM_WORKERS = NUM_CORES * NUM_SUBCORES
CHUNK = 200
NBUF = 4


def _gather_body(n_per_w, x_hbm, te_hbm, out_hbm, idx_v, rows_v, gsem, wsem):
    nchunks = n_per_w // CHUNK
    wid = lax.axis_index("s") * NUM_CORES + lax.axis_index("c")
    base = wid * n_per_w
    row0 = wid * nchunks
    pltpu.sync_copy(x_hbm.at[pl.ds(base, n_per_w)], idx_v)

    def idx_slice(g):
        return idx_v.at[pl.ds(g * CHUNK, CHUNK)]

    def out_slice(g):
        return out_hbm.at[row0 + g]

    def start_gather(g, b):
        pltpu.async_copy(te_hbm.at[idx_slice(g)], rows_v.at[b], gsem.at[b])

    def wait_gather(g, b):
        pltpu.make_async_copy(te_hbm.at[idx_slice(g)], rows_v.at[b],
                              gsem.at[b]).wait()

    def start_wo(g, b):
        pltpu.async_copy(rows_v.at[b], out_slice(g), wsem.at[b])

    def wait_wo(g, b):
        pltpu.make_async_copy(rows_v.at[b], out_slice(g), wsem.at[b]).wait()

    for b in range(NBUF):
        start_gather(b, b)
    for g in (0, 1):
        wait_gather(g, g)
        start_wo(g, g)
    for g in (2, 3):
        wait_gather(g, g)
        start_wo(g, g)
        wait_wo(g - 2, g - 2)
        start_gather(g + 2, g - 2)

    def outer(go, carry):
        for b in range(NBUF):
            g = go * NBUF + b
            wait_gather(g, b)
            start_wo(g, b)
            b2 = (b + 2) % NBUF
            wait_wo(g - 2, b2)
            start_gather(g + 2, b2)
        return carry

    lax.fori_loop(1, nchunks // NBUF - 1, outer, 0, unroll=False)

    for b in range(NBUF):
        g = nchunks - NBUF + b
        wait_gather(g, b)
        start_wo(g, b)
        if b < 2:
            b2 = (b + 2) % NBUF
            wait_wo(g - 2, b2)
            start_gather(g + 2, b2)
    for b in range(NBUF):
        wait_wo(nchunks - NBUF + b, b)


def kernel(x, te):
    batch, seq = x.shape
    n_total = batch * seq
    n_per_w = n_total // NUM_WORKERS
    assert n_per_w % (NBUF * CHUNK) == 0

    assert seq == CHUNK

    x_flat = x.reshape(n_total).astype(jnp.int32)

    mesh = plsc.VectorSubcoreMesh(core_axis_name="c", subcore_axis_name="s")
    run = pl.kernel(
        functools.partial(_gather_body, n_per_w),
        out_type=jax.ShapeDtypeStruct((batch, seq, D_MODEL), jnp.float32),
        mesh=mesh,
        scratch_types=[
            pltpu.VMEM((n_per_w,), jnp.int32),
            pltpu.VMEM((NBUF, CHUNK, D_MODEL), jnp.float32),
            pltpu.SemaphoreType.DMA((NBUF,)),
            pltpu.SemaphoreType.DMA((NBUF,)),
        ],
        compiler_params=pltpu.CompilerParams(use_tc_tiling_on_sc=False),
    )
    return run(x_flat, te)
